# baseline (speedup 1.0000x reference)
import functools

import jax
import jax.numpy as jnp
from jax import lax
from jax.experimental import pallas as pl
from jax.experimental.pallas import tpu as pltpu
from jax.experimental.pallas import tpu_sc as plsc

DIM = 768
ROWS = 4096
NC, NS = 2, 16
NW = NC * NS
PER_W = ROWS // NW
CHUNK = 32
NCH = PER_W // CHUNK
NTAB = 3
NBUF = 4


def _build(dual):
    nout = 2 * NTAB if dual else NTAB
    mesh = plsc.VectorSubcoreMesh(core_axis_name="c", subcore_axis_name="s")
    out_type = [jax.ShapeDtypeStruct((ROWS, DIM), jnp.float32)] * nout
    scratch = (
        [pltpu.VMEM((NCH, CHUNK), jnp.int32)]
        + [pltpu.VMEM((CHUNK, DIM), jnp.float32)] * NBUF
        + [pltpu.SemaphoreType.DMA] * (2 * NBUF)
    )

    @functools.partial(pl.kernel, mesh=mesh, out_type=out_type,
                       scratch_types=scratch)
    def gather(idx_hbm, t0, t1, t2, *rest):
        tabs = [t0, t1, t2]
        outs = list(rest[:nout])
        idx_v = rest[nout]
        bufs = list(rest[nout + 1:nout + 1 + NBUF])
        gsems = list(rest[nout + 1 + NBUF:nout + 1 + 2 * NBUF])
        ssems = list(rest[nout + 1 + 2 * NBUF:])
        wid = lax.axis_index("s") * NC + lax.axis_index("c")
        base = wid * PER_W
        pltpu.sync_copy(idx_hbm.at[pl.ds(wid * NCH, NCH)], idx_v)

        total = NTAB * NCH
        ghandles = {}
        shandles = {}

        def start_gather(c):
            t, h = divmod(c, NCH)
            b = c % NBUF
            ghandles[c] = pltpu.async_copy(
                tabs[t].at[idx_v.at[h]], bufs[b], gsems[b])

        def start_scatter(c):
            t, h = divmod(c, NCH)
            b = c % NBUF
            dst = pl.ds(base + h * CHUNK, CHUNK)
            hs = [pltpu.async_copy(bufs[b], outs[t].at[dst], ssems[b])]
            if dual:
                hs.append(pltpu.async_copy(
                    bufs[b], outs[2 * NTAB - 1 - t].at[dst], ssems[b]))
            shandles[c] = hs

        def wait_scatter(c):
            for h in shandles[c]:
                h.wait()

        for c in range(NBUF - 1):
            start_gather(c)
        for c in range(total):
            ghandles[c].wait()
            start_scatter(c)
            n = c + NBUF - 1
            if n < total:
                if n >= NBUF:
                    wait_scatter(n - NBUF)
                start_gather(n)
        for c in range(total - NBUF, total):
            wait_scatter(c)

    return gather


_GATHER_SINGLE = _build(dual=False)
_GATHER_DUAL = _build(dual=True)


def kernel(inputs, W0, W1, W2, W3, W4, W5):
    b, s = inputs.shape
    idx = inputs.reshape(NW * NCH, CHUNK).astype(jnp.int32)
    a0, a1, a2 = _GATHER_SINGLE(idx, W0, W1, W2)
    o3, o4, o5, o6, o7, o8 = _GATHER_DUAL(idx, W3, W4, W5)
    outs = [a0, a1, a2, o3, o4, o5, o6, o7, o8, a2, a1, a0]
    return tuple(o.reshape(b, s, DIM) for o in outs)


# speedup vs baseline: 1.0780x; 1.0780x over previous
import functools

import jax
import jax.numpy as jnp
from jax import lax
from jax.experimental import pallas as pl
from jax.experimental.pallas import tpu as pltpu
from jax.experimental.pallas import tpu_sc as plsc

DIM = 768
ROWS = 4096
NC, NS = 2, 16
NW = NC * NS
PER_W = ROWS // NW
CHUNK = 32
NCH = PER_W // CHUNK
NTAB = 3
NBUF = 4


def _build(dual):
    nout = 2 * NTAB if dual else NTAB
    mesh = plsc.VectorSubcoreMesh(core_axis_name="c", subcore_axis_name="s")
    out_type = [jax.ShapeDtypeStruct((ROWS, DIM), jnp.float32)] * nout
    scratch = (
        [pltpu.VMEM((NCH, CHUNK), jnp.int32)]
        + [pltpu.VMEM((CHUNK, DIM), jnp.float32)] * NBUF
        + [pltpu.SemaphoreType.DMA] * (2 * NBUF)
    )

    @functools.partial(pl.kernel, mesh=mesh, out_type=out_type,
                       scratch_types=scratch)
    def gather(idx_hbm, t0, t1, t2, *rest):
        tabs = [t0, t1, t2]
        outs = list(rest[:nout])
        idx_v = rest[nout]
        bufs = list(rest[nout + 1:nout + 1 + NBUF])
        gsems = list(rest[nout + 1 + NBUF:nout + 1 + 2 * NBUF])
        ssems = list(rest[nout + 1 + 2 * NBUF:])
        wid = lax.axis_index("s") * NC + lax.axis_index("c")
        base = wid * PER_W
        pltpu.sync_copy(idx_hbm.at[pl.ds(wid * NCH, NCH)], idx_v)

        total = NTAB * NCH
        ghandles = {}
        shandles = {}

        def start_gather(c):
            t, h = divmod(c, NCH)
            b = c % NBUF
            ghandles[c] = pltpu.async_copy(
                tabs[t].at[idx_v.at[h]], bufs[b], gsems[b])

        def start_scatter(c):
            t, h = divmod(c, NCH)
            b = c % NBUF
            dst = pl.ds(base + h * CHUNK, CHUNK)
            hs = [pltpu.async_copy(bufs[b], outs[t].at[dst], ssems[b])]
            if dual:
                hs.append(pltpu.async_copy(
                    bufs[b], outs[2 * NTAB - 1 - t].at[dst], ssems[b]))
            shandles[c] = hs

        def wait_scatter(c):
            for h in shandles[c]:
                h.wait()

        for c in range(NBUF - 1):
            start_gather(c)
        for c in range(total):
            ghandles[c].wait()
            start_scatter(c)
            n = c + NBUF - 1
            if n < total:
                if n >= NBUF:
                    wait_scatter(n - NBUF)
                start_gather(n)
        for c in range(total - NBUF, total):
            wait_scatter(c)

    return gather


_GATHER_SINGLE = _build(dual=False)
_GATHER_DUAL = _build(dual=True)


def kernel(inputs, W0, W1, W2, W3, W4, W5):
    b, s = inputs.shape
    idx = inputs.reshape(NW * NCH, CHUNK).astype(jnp.int32)
    a0, a1, a2 = _GATHER_SINGLE(idx, W0, W1, W2)
    o3, o4, o5, o6, o7, o8 = _GATHER_DUAL(idx, W3, W4, W5)
    one = 1.0 + 0.0 * idx[0, 0].astype(jnp.float32)
    d2, d1, d0 = a2 * one, a1 * one, a0 * one
    outs = [a0, a1, a2, o3, o4, o5, o6, o7, o8, d2, d1, d0]
    return tuple(o.reshape(b, s, DIM) for o in outs)


# speedup vs baseline: 1.1384x; 1.0560x over previous
import functools

import jax
import jax.numpy as jnp
from jax import lax
from jax.experimental import pallas as pl
from jax.experimental.pallas import tpu as pltpu
from jax.experimental.pallas import tpu_sc as plsc

DIM = 768
ROWS = 4096
NC, NS = 2, 16
NW = NC * NS
PER_W = ROWS // NW
CHUNK = 32
NCH = PER_W // CHUNK
NBUF = 4
KA = 2


def _build(ntab, dual):
    nout = 2 * ntab if dual else ntab
    mesh = plsc.VectorSubcoreMesh(core_axis_name="c", subcore_axis_name="s")
    out_type = [jax.ShapeDtypeStruct((ROWS, DIM), jnp.float32)] * nout
    scratch = (
        [pltpu.VMEM((NCH, CHUNK), jnp.int32)]
        + [pltpu.VMEM((CHUNK, DIM), jnp.float32)] * NBUF
        + [pltpu.SemaphoreType.DMA] * (2 * NBUF)
    )

    @functools.partial(pl.kernel, mesh=mesh, out_type=out_type,
                       scratch_types=scratch)
    def gather(idx_hbm, *rest):
        tabs = list(rest[:ntab])
        outs = list(rest[ntab:ntab + nout])
        idx_v = rest[ntab + nout]
        bufs = list(rest[ntab + nout + 1:ntab + nout + 1 + NBUF])
        gsems = list(rest[ntab + nout + 1 + NBUF:ntab + nout + 1 + 2 * NBUF])
        ssems = list(rest[ntab + nout + 1 + 2 * NBUF:])
        wid = lax.axis_index("s") * NC + lax.axis_index("c")
        base = wid * PER_W
        pltpu.sync_copy(idx_hbm.at[pl.ds(wid * NCH, NCH)], idx_v)

        total = ntab * NCH
        ghandles = {}
        shandles = {}

        def start_gather(c):
            t, h = divmod(c, NCH)
            b = c % NBUF
            ghandles[c] = pltpu.async_copy(
                tabs[t].at[idx_v.at[h]], bufs[b], gsems[b])

        def start_scatter(c):
            t, h = divmod(c, NCH)
            b = c % NBUF
            dst = pl.ds(base + h * CHUNK, CHUNK)
            hs = [pltpu.async_copy(bufs[b], outs[t].at[dst], ssems[b])]
            if dual:
                hs.append(pltpu.async_copy(
                    bufs[b], outs[ntab + t].at[dst], ssems[b]))
            shandles[c] = hs

        def wait_scatter(c):
            for h in shandles[c]:
                h.wait()

        for c in range(NBUF - 1):
            start_gather(c)
        for c in range(total):
            ghandles[c].wait()
            start_scatter(c)
            n = c + NBUF - 1
            if n < total:
                if n >= NBUF:
                    wait_scatter(n - NBUF)
                start_gather(n)
        for c in range(total - NBUF, total):
            wait_scatter(c)

    return gather


_GATHER_SINGLE = _build(KA, dual=False)
_GATHER_DUAL = _build(6 - KA, dual=True)


def kernel(inputs, W0, W1, W2, W3, W4, W5):
    b, s = inputs.shape
    tables = [W0, W1, W2, W3, W4, W5]
    idx = inputs.reshape(NW * NCH, CHUNK).astype(jnp.int32)
    a = list(_GATHER_SINGLE(idx, *tables[:KA]))
    bo = list(_GATHER_DUAL(idx, *tables[KA:]))
    nb = 6 - KA
    one = 1.0 + 0.0 * idx[0, 0].astype(jnp.float32)
    outs = [None] * 12
    for t in range(KA):
        outs[t] = a[t]
        outs[11 - t] = a[t] * one
    for j in range(nb):
        outs[KA + j] = bo[j]
        outs[11 - KA - j] = bo[nb + j]
    return tuple(o.reshape(b, s, DIM) for o in outs)


# speedup vs baseline: 1.1549x; 1.0145x over previous
import functools

import jax
import jax.numpy as jnp
from jax import lax
from jax.experimental import pallas as pl
from jax.experimental.pallas import tpu as pltpu
from jax.experimental.pallas import tpu_sc as plsc

DIM = 768
ROWS = 4096
NC, NS = 2, 16
NW = NC * NS
PER_W = ROWS // NW
CHUNK = 32
NCH = PER_W // CHUNK
NBUF = 4
KA = 1


def _build(ntab, dual):
    nout = 2 * ntab if dual else ntab
    mesh = plsc.VectorSubcoreMesh(core_axis_name="c", subcore_axis_name="s")
    out_type = [jax.ShapeDtypeStruct((ROWS, DIM), jnp.float32)] * nout
    scratch = (
        [pltpu.VMEM((NCH, CHUNK), jnp.int32)]
        + [pltpu.VMEM((CHUNK, DIM), jnp.float32)] * NBUF
        + [pltpu.SemaphoreType.DMA] * (2 * NBUF)
    )

    @functools.partial(pl.kernel, mesh=mesh, out_type=out_type,
                       scratch_types=scratch)
    def gather(idx_hbm, *rest):
        tabs = list(rest[:ntab])
        outs = list(rest[ntab:ntab + nout])
        idx_v = rest[ntab + nout]
        bufs = list(rest[ntab + nout + 1:ntab + nout + 1 + NBUF])
        gsems = list(rest[ntab + nout + 1 + NBUF:ntab + nout + 1 + 2 * NBUF])
        ssems = list(rest[ntab + nout + 1 + 2 * NBUF:])
        wid = lax.axis_index("s") * NC + lax.axis_index("c")
        base = wid * PER_W
        pltpu.sync_copy(idx_hbm.at[pl.ds(wid * NCH, NCH)], idx_v)

        total = ntab * NCH
        ghandles = {}
        shandles = {}

        def start_gather(c):
            t, h = divmod(c, NCH)
            b = c % NBUF
            ghandles[c] = pltpu.async_copy(
                tabs[t].at[idx_v.at[h]], bufs[b], gsems[b])

        def start_scatter(c):
            t, h = divmod(c, NCH)
            b = c % NBUF
            dst = pl.ds(base + h * CHUNK, CHUNK)
            hs = [pltpu.async_copy(bufs[b], outs[t].at[dst], ssems[b])]
            if dual:
                hs.append(pltpu.async_copy(
                    bufs[b], outs[ntab + t].at[dst], ssems[b]))
            shandles[c] = hs

        def wait_scatter(c):
            for h in shandles[c]:
                h.wait()

        for c in range(NBUF - 1):
            start_gather(c)
        for c in range(total):
            ghandles[c].wait()
            start_scatter(c)
            n = c + NBUF - 1
            if n < total:
                if n >= NBUF:
                    wait_scatter(n - NBUF)
                start_gather(n)
        for c in range(total - NBUF, total):
            wait_scatter(c)

    return gather


_GATHER_SINGLE = _build(KA, dual=False)
_GATHER_DUAL = _build(6 - KA, dual=True)


def kernel(inputs, W0, W1, W2, W3, W4, W5):
    b, s = inputs.shape
    tables = [W0, W1, W2, W3, W4, W5]
    idx = inputs.reshape(NW * NCH, CHUNK).astype(jnp.int32)
    a = list(_GATHER_SINGLE(idx, *tables[:KA]))
    bo = list(_GATHER_DUAL(idx, *tables[KA:]))
    nb = 6 - KA
    one = 1.0 + 0.0 * idx[0, 0].astype(jnp.float32)
    outs = [None] * 12
    for t in range(KA):
        outs[t] = a[t]
        outs[11 - t] = a[t] * one
    for j in range(nb):
        outs[KA + j] = bo[j]
        outs[11 - KA - j] = bo[nb + j]
    return tuple(o.reshape(b, s, DIM) for o in outs)


# speedup vs baseline: 1.1705x; 1.0135x over previous
import functools

import jax
import jax.numpy as jnp
from jax import lax
from jax.experimental import pallas as pl
from jax.experimental.pallas import tpu as pltpu
from jax.experimental.pallas import tpu_sc as plsc

DIM = 768
ROWS = 4096
NC, NS = 2, 16
NW = NC * NS
PER_W = ROWS // NW
CHUNK = 32
NCH = PER_W // CHUNK
NTAB = 6
NBUF = 4


def _build(seq):
    mesh = plsc.VectorSubcoreMesh(core_axis_name="c", subcore_axis_name="s")
    out_type = [jax.ShapeDtypeStruct((ROWS, DIM), jnp.float32)] * (2 * NTAB)
    scratch = (
        [pltpu.VMEM((NCH, CHUNK), jnp.int32)]
        + [pltpu.VMEM((CHUNK, DIM), jnp.float32)] * NBUF
        + [pltpu.SemaphoreType.DMA] * (2 * NBUF)
    )

    @functools.partial(pl.kernel, mesh=mesh, out_type=out_type,
                       scratch_types=scratch)
    def gather12(idx_hbm, t0, t1, t2, t3, t4, t5, *rest):
        tabs = [t0, t1, t2, t3, t4, t5]
        outs = list(rest[:2 * NTAB])
        idx_v = rest[2 * NTAB]
        bufs = list(rest[2 * NTAB + 1:2 * NTAB + 1 + NBUF])
        gsems = list(rest[2 * NTAB + 1 + NBUF:2 * NTAB + 1 + 2 * NBUF])
        ssems = list(rest[2 * NTAB + 1 + 2 * NBUF:])
        wid = lax.axis_index("s") * NC + lax.axis_index("c")
        base = wid * PER_W
        row = wid // (seq // PER_W)
        col = (wid % (seq // PER_W)) * PER_W
        for h in range(NCH):
            pltpu.sync_copy(
                idx_hbm.at[row, pl.ds(col + h * CHUNK, CHUNK)], idx_v.at[h])

        total = NTAB * NCH
        ghandles = {}
        shandles = {}

        def start_gather(c):
            t, h = divmod(c, NCH)
            b = c % NBUF
            ghandles[c] = pltpu.async_copy(
                tabs[t].at[idx_v.at[h]], bufs[b], gsems[b])

        def start_scatter(c):
            t, h = divmod(c, NCH)
            b = c % NBUF
            dst = pl.ds(base + h * CHUNK, CHUNK)
            shandles[c] = (
                pltpu.async_copy(bufs[b], outs[t].at[dst], ssems[b]),
                pltpu.async_copy(bufs[b], outs[11 - t].at[dst], ssems[b]),
            )

        def wait_scatter(c):
            shandles[c][0].wait()
            shandles[c][1].wait()

        for c in range(NBUF - 1):
            start_gather(c)
        for c in range(total):
            ghandles[c].wait()
            start_scatter(c)
            n = c + NBUF - 1
            if n < total:
                if n >= NBUF:
                    wait_scatter(n - NBUF)
                start_gather(n)
        for c in range(total - NBUF, total):
            wait_scatter(c)

    return gather12


@functools.cache
def _gather12(seq):
    return _build(seq)


def kernel(inputs, W0, W1, W2, W3, W4, W5):
    b, s = inputs.shape
    outs = _gather12(s)(inputs.astype(jnp.int32), W0, W1, W2, W3, W4, W5)
    return tuple(o.reshape(b, s, DIM) for o in outs)


# speedup vs baseline: 1.1706x; 1.0001x over previous
import functools

import jax
import jax.numpy as jnp
from jax import lax
from jax.experimental import pallas as pl
from jax.experimental.pallas import tpu as pltpu
from jax.experimental.pallas import tpu_sc as plsc

DIM = 768
ROWS = 4096
NC, NS = 2, 16
NW = NC * NS
PER_W = ROWS // NW
CHUNK = 64
NCH = PER_W // CHUNK
NTAB = 6
NBUF = 2


def _build(seq):
    mesh = plsc.VectorSubcoreMesh(core_axis_name="c", subcore_axis_name="s")
    out_type = [jax.ShapeDtypeStruct((ROWS, DIM), jnp.float32)] * (2 * NTAB)
    scratch = (
        [pltpu.VMEM((NCH, CHUNK), jnp.int32)]
        + [pltpu.VMEM((CHUNK, DIM), jnp.float32)] * NBUF
        + [pltpu.SemaphoreType.DMA] * (2 * NBUF)
    )

    @functools.partial(pl.kernel, mesh=mesh, out_type=out_type,
                       scratch_types=scratch)
    def gather12(idx_hbm, t0, t1, t2, t3, t4, t5, *rest):
        tabs = [t0, t1, t2, t3, t4, t5]
        outs = list(rest[:2 * NTAB])
        idx_v = rest[2 * NTAB]
        bufs = list(rest[2 * NTAB + 1:2 * NTAB + 1 + NBUF])
        gsems = list(rest[2 * NTAB + 1 + NBUF:2 * NTAB + 1 + 2 * NBUF])
        ssems = list(rest[2 * NTAB + 1 + 2 * NBUF:])
        wid = lax.axis_index("s") * NC + lax.axis_index("c")
        base = wid * PER_W
        row = wid // (seq // PER_W)
        col = (wid % (seq // PER_W)) * PER_W
        for h in range(NCH):
            pltpu.sync_copy(
                idx_hbm.at[row, pl.ds(col + h * CHUNK, CHUNK)], idx_v.at[h])

        total = NTAB * NCH
        ghandles = {}
        shandles = {}

        def start_gather(c):
            t, h = divmod(c, NCH)
            b = c % NBUF
            ghandles[c] = pltpu.async_copy(
                tabs[t].at[idx_v.at[h]], bufs[b], gsems[b])

        def start_scatter(c):
            t, h = divmod(c, NCH)
            b = c % NBUF
            dst = pl.ds(base + h * CHUNK, CHUNK)
            shandles[c] = (
                pltpu.async_copy(bufs[b], outs[t].at[dst], ssems[b]),
                pltpu.async_copy(bufs[b], outs[11 - t].at[dst], ssems[b]),
            )

        def wait_scatter(c):
            shandles[c][0].wait()
            shandles[c][1].wait()

        for c in range(NBUF - 1):
            start_gather(c)
        for c in range(total):
            ghandles[c].wait()
            start_scatter(c)
            n = c + NBUF - 1
            if n < total:
                if n >= NBUF:
                    wait_scatter(n - NBUF)
                start_gather(n)
        for c in range(total - NBUF, total):
            wait_scatter(c)

    return gather12


@functools.cache
def _gather12(seq):
    return _build(seq)


def kernel(inputs, W0, W1, W2, W3, W4, W5):
    b, s = inputs.shape
    outs = _gather12(s)(inputs.astype(jnp.int32), W0, W1, W2, W3, W4, W5)
    return tuple(o.reshape(b, s, DIM) for o in outs)
